# baseline (speedup 1.0000x reference)
import jax
import jax.numpy as jnp
from jax import lax
from jax.experimental import pallas as pl
from jax.experimental.pallas import tpu as pltpu
from jax.experimental.pallas import tpu_sc as plsc

BATCH = 16384
DIM = 64
NC, NS, L = 2, 16, 16
NW = NC * NS
B_PER_W = BATCH // NW
CHUNK = 128
NCHUNK = B_PER_W // CHUNK
GROUPS = CHUNK // L
FG = DIM // L
ENT = 1_000_000


def _body(heads, rels, tails, ent_re_p, ent_im_p, rel_cat, out,
          idx_h, idx_r, idx_t, idx_h2, idx_t2,
          hrv, hiv, trv, tiv, rv, srow, out_v, sem):
    wid = lax.axis_index("s") * NC + lax.axis_index("c")
    lane = lax.iota(jnp.int32, L)

    def chunk_body(c, _):
        base = wid * B_PER_W + c * CHUNK
        pltpu.sync_copy(heads.at[pl.ds(base, CHUNK)], idx_h)
        pltpu.sync_copy(rels.at[pl.ds(base, CHUNK)], idx_r)
        pltpu.sync_copy(tails.at[pl.ds(base, CHUNK)], idx_t)
        for srcv, dstv in ((idx_h, idx_h2), (idx_t, idx_t2)):
            for k in range(GROUPS):
                dstv[pl.ds(k * L, L)] = srcv[pl.ds(k * L, L)] >> 1
        cps = [
            pltpu.async_copy(ent_re_p.at[idx_h2], hrv, sem),
            pltpu.async_copy(ent_im_p.at[idx_h2], hiv, sem),
            pltpu.async_copy(ent_re_p.at[idx_t2], trv, sem),
            pltpu.async_copy(ent_im_p.at[idx_t2], tiv, sem),
            pltpu.async_copy(rel_cat.at[idx_r], rv, sem),
        ]
        for cp in cps:
            cp.wait()

        def group_body(g, _):
            evh = idx_h[pl.ds(g * L, L)]
            evt = idx_t[pl.ds(g * L, L)]
            for j in range(L):
                ph = (evh[j] & 1) * DIM
                pt = (evt[j] & 1) * DIM
                acc = None
                for k in range(FG):
                    f = pl.ds(k * L, L)
                    fi = pl.ds(DIM + k * L, L)
                    hr = hrv[g * L + j, pl.ds(ph + k * L, L)]
                    hi = hiv[g * L + j, pl.ds(ph + k * L, L)]
                    tr = trv[g * L + j, pl.ds(pt + k * L, L)]
                    ti = tiv[g * L + j, pl.ds(pt + k * L, L)]
                    rr = rv[g * L + j, f]
                    ri = rv[g * L + j, fi]
                    term = (hr * (rr * tr + ri * ti)
                            + hi * (rr * ti - ri * tr))
                    acc = term if acc is None else acc + term
                srow[j] = acc
            tot = jnp.zeros((L,), jnp.float32)
            for l in range(L):
                tot = tot + plsc.load_gather(
                    srow, [lane, jnp.full((L,), l, jnp.int32)])
            out_v[pl.ds(c * CHUNK + g * L, L)] = tot
            return ()

        lax.fori_loop(0, GROUPS, group_body, ())
        return ()

    lax.fori_loop(0, NCHUNK, chunk_body, ())
    pltpu.sync_copy(out_v, out.at[pl.ds(wid * B_PER_W, B_PER_W)])


@jax.jit
def _complex_score(heads, relations, tails, entity_re, entity_im,
                   relation_re, relation_im):
    ent_re_p = entity_re.reshape(ENT // 2, 2 * DIM)
    ent_im_p = entity_im.reshape(ENT // 2, 2 * DIM)
    rel_cat = jnp.concatenate([relation_re, relation_im], axis=1)
    mesh = plsc.VectorSubcoreMesh(core_axis_name="c", subcore_axis_name="s",
                                  num_cores=NC, num_subcores=NS)
    kern = pl.kernel(
        _body,
        out_type=jax.ShapeDtypeStruct((BATCH,), jnp.float32),
        mesh=mesh,
        scratch_types=[
            pltpu.VMEM((CHUNK,), jnp.int32),
            pltpu.VMEM((CHUNK,), jnp.int32),
            pltpu.VMEM((CHUNK,), jnp.int32),
            pltpu.VMEM((CHUNK,), jnp.int32),
            pltpu.VMEM((CHUNK,), jnp.int32),
            pltpu.VMEM((CHUNK, 2 * DIM), jnp.float32),
            pltpu.VMEM((CHUNK, 2 * DIM), jnp.float32),
            pltpu.VMEM((CHUNK, 2 * DIM), jnp.float32),
            pltpu.VMEM((CHUNK, 2 * DIM), jnp.float32),
            pltpu.VMEM((CHUNK, 2 * DIM), jnp.float32),
            pltpu.VMEM((L, L), jnp.float32),
            pltpu.VMEM((B_PER_W,), jnp.float32),
            pltpu.SemaphoreType.DMA,
        ],
        compiler_params=pltpu.CompilerParams(needs_layout_passes=False),
        name="complex_score_sc",
    )
    return kern(heads, relations, tails, ent_re_p, ent_im_p, rel_cat)


def kernel(heads, relations, tails, entity_re, entity_im, relation_re,
           relation_im):
    return _complex_score(
        heads.astype(jnp.int32), relations.astype(jnp.int32),
        tails.astype(jnp.int32), entity_re, entity_im,
        relation_re, relation_im)


# speedup vs baseline: 1.2138x; 1.2138x over previous
import jax
import jax.numpy as jnp
from jax import lax
from jax.experimental import pallas as pl
from jax.experimental.pallas import tpu as pltpu
from jax.experimental.pallas import tpu_sc as plsc

BATCH = 16384
DIM = 64
NC, NS, L = 2, 16, 16
NW = NC * NS
B_PER_W = BATCH // NW
CHUNK = 128
NCHUNK = B_PER_W // CHUNK
GROUPS = CHUNK // L
FG = DIM // L


def _body(heads, rels, tails, ent_cat, rel_cat, out,
          idx_h, idx_r, idx_t, hv, tv, rv, srow, out_v, sem):
    wid = lax.axis_index("s") * NC + lax.axis_index("c")
    lane = lax.iota(jnp.int32, L)

    def chunk_body(c, _):
        base = wid * B_PER_W + c * CHUNK
        pltpu.sync_copy(heads.at[pl.ds(base, CHUNK)], idx_h)
        pltpu.sync_copy(rels.at[pl.ds(base, CHUNK)], idx_r)
        pltpu.sync_copy(tails.at[pl.ds(base, CHUNK)], idx_t)
        cps = [
            pltpu.async_copy(ent_cat.at[idx_h], hv, sem),
            pltpu.async_copy(ent_cat.at[idx_t], tv, sem),
            pltpu.async_copy(rel_cat.at[idx_r], rv, sem),
        ]
        for cp in cps:
            cp.wait()

        def group_body(g, _):
            for j in range(L):
                acc = None
                for k in range(FG):
                    f = pl.ds(k * L, L)
                    fi = pl.ds(DIM + k * L, L)
                    hr = hv[g * L + j, f]
                    hi = hv[g * L + j, fi]
                    tr = tv[g * L + j, f]
                    ti = tv[g * L + j, fi]
                    rr = rv[g * L + j, f]
                    ri = rv[g * L + j, fi]
                    term = (hr * (rr * tr + ri * ti)
                            + hi * (rr * ti - ri * tr))
                    acc = term if acc is None else acc + term
                srow[j] = acc
            tot = jnp.zeros((L,), jnp.float32)
            for l in range(L):
                tot = tot + plsc.load_gather(
                    srow, [lane, jnp.full((L,), l, jnp.int32)])
            out_v[pl.ds(c * CHUNK + g * L, L)] = tot
            return ()

        lax.fori_loop(0, GROUPS, group_body, ())
        return ()

    lax.fori_loop(0, NCHUNK, chunk_body, ())
    pltpu.sync_copy(out_v, out.at[pl.ds(wid * B_PER_W, B_PER_W)])


@jax.jit
def _complex_score(heads, relations, tails, entity_re, entity_im,
                   relation_re, relation_im):
    ent_cat = jnp.concatenate([entity_re, entity_im], axis=1)
    rel_cat = jnp.concatenate([relation_re, relation_im], axis=1)
    mesh = plsc.VectorSubcoreMesh(core_axis_name="c", subcore_axis_name="s",
                                  num_cores=NC, num_subcores=NS)
    kern = pl.kernel(
        _body,
        out_type=jax.ShapeDtypeStruct((BATCH,), jnp.float32),
        mesh=mesh,
        scratch_types=[
            pltpu.VMEM((CHUNK,), jnp.int32),
            pltpu.VMEM((CHUNK,), jnp.int32),
            pltpu.VMEM((CHUNK,), jnp.int32),
            pltpu.VMEM((CHUNK, 2 * DIM), jnp.float32),
            pltpu.VMEM((CHUNK, 2 * DIM), jnp.float32),
            pltpu.VMEM((CHUNK, 2 * DIM), jnp.float32),
            pltpu.VMEM((L, L), jnp.float32),
            pltpu.VMEM((B_PER_W,), jnp.float32),
            pltpu.SemaphoreType.DMA,
        ],
        compiler_params=pltpu.CompilerParams(needs_layout_passes=False),
        name="complex_score_sc",
    )
    return kern(heads, relations, tails, ent_cat, rel_cat)


def kernel(heads, relations, tails, entity_re, entity_im, relation_re,
           relation_im):
    return _complex_score(
        heads.astype(jnp.int32), relations.astype(jnp.int32),
        tails.astype(jnp.int32), entity_re, entity_im,
        relation_re, relation_im)
